# baseline (speedup 1.0000x reference)
import jax
import jax.numpy as jnp
from jax.experimental import pallas as pl
from jax.experimental.pallas import tpu as pltpu

_N_EVENTS = 4096
_N_NEURONS = 1024
_TS = 256


def _body(all_ts_ref, w_in_ref, ch_in_ref, out_ref, w_ref, ch_ref,
          wn2_ref, rinv_ref):
    w_ref[...] = w_in_ref[...]
    ch_ref[...] = ch_in_ref[...]
    wn2_ref[...] = jnp.sum(w_in_ref[...] * w_in_ref[...], axis=1)
    rinv_ref[...] = jax.lax.rsqrt(wn2_ref[...])
    chsum0 = jnp.sum(ch_in_ref[...])
    iota_n = jax.lax.iota(jnp.int32, _N_NEURONS)

    def step(t, carry):
        tsd, gain8 = carry
        gain = gain8.reshape(_N_NEURONS)
        w = w_ref[...]
        mv = jax.lax.dot_general(
            tsd, w, (((1,), (1,)), ((), ())),
            precision=jax.lax.Precision.DEFAULT,
            preferred_element_type=jnp.float32).reshape(_N_NEURONS)
        beta = mv * rinv_ref[...]
        ch = ch_ref[...]
        bh = gain * beta
        alpha_v = jnp.float32(0.01) / (1.0 + ch * jnp.float32(5e-5))
        acand = alpha_v * beta
        n = jnp.argmax(bh).astype(jnp.int32)
        onehot = iota_n == n
        a = jnp.sum(jnp.where(onehot, acand, 0.0))
        ck = w_ref[pl.ds(n, 1), :]
        newrow = ck + a * (tsd - ck)
        w_ref[pl.ds(n, 1), :] = newrow
        wn2_new = jnp.sum(newrow * newrow)
        wn2_ref[...] = jnp.where(onehot, wn2_new, wn2_ref[...])
        rinv_ref[...] = jnp.where(onehot, jnp.max(jax.lax.rsqrt(
            jnp.full((8, 128), wn2_new, jnp.float32))), rinv_ref[...])
        ch_up = jnp.where(onehot, ch + 1.0, ch)
        ch_ref[...] = ch_up
        out_ref[0] = n
        t_nx = jnp.minimum(t + 1, _N_EVENTS - 1)
        ts_nx = all_ts_ref[pl.ds(t_nx, 1), :]
        tsd_nx = ts_nx / jnp.sqrt(jnp.sum(ts_nx * ts_nx))
        chsum_nx = chsum0 + (t + 1).astype(jnp.float32)
        gain_nx = jnp.exp((1.0 - (ch_up * 1024.0) / chsum_nx) * 0.25)
        return (tsd_nx, gain_nx.reshape(8, 128))

    ts_0 = all_ts_ref[pl.ds(0, 1), :]
    tsd_0 = ts_0 / jnp.sqrt(jnp.sum(ts_0 * ts_0))
    gain_0 = jnp.exp((1.0 - (ch_in_ref[...] * 1024.0) / chsum0) * 0.25)
    jax.lax.fori_loop(0, _N_EVENTS, step, (tsd_0, gain_0.reshape(8, 128)))


def kernel(all_ts, W, cumhisto):
    out = pl.pallas_call(
        _body,
        out_shape=jax.ShapeDtypeStruct((1,), jnp.int32),
        in_specs=[
            pl.BlockSpec(memory_space=pltpu.VMEM),
            pl.BlockSpec(memory_space=pltpu.VMEM),
            pl.BlockSpec(memory_space=pltpu.VMEM),
        ],
        out_specs=pl.BlockSpec(memory_space=pltpu.SMEM),
        scratch_shapes=[
            pltpu.VMEM((_N_NEURONS, _TS), jnp.float32),
            pltpu.VMEM((_N_NEURONS,), jnp.float32),
            pltpu.VMEM((_N_NEURONS,), jnp.float32),
            pltpu.VMEM((_N_NEURONS,), jnp.float32),
        ],
    )(all_ts, W, cumhisto)
    return out[0]
